# baseline (speedup 1.0000x reference)
import jax
import jax.numpy as jnp
from jax import lax
from jax.experimental import pallas as pl
from jax.experimental.pallas import tpu as pltpu
from jax.experimental.pallas import tpu_sc as plsc

_NT = 16
_NC = 2
_L = 16


def _make_sc_kernel(G, Gp, Ep):
    GPT = Gp // _NT
    EPT = Ep // _NT
    EPT2 = EPT // _NC
    mesh = plsc.VectorSubcoreMesh(
        core_axis_name="c", subcore_axis_name="s", num_cores=_NC)

    def body(ei_h, x_h, t2_h, c2_h, dinv_h, x0p_h,
             src_v, dst_v, x0_v, dinv_v, hist_v, t_v, c_v, red_v, red2_v,
             o1_v, o2_v,
             sem_e, sem_x, sem_red, sem_out, sh_a, sh_b, sh_dinv):
        cid = lax.axis_index("c")
        sid = lax.axis_index("s")
        gbase = sid * GPT
        ebase = sid * EPT

        with jax.named_scope("sc_stage"):
            cp_s = pltpu.async_copy(
                ei_h.at[pl.ds(ebase + cid * EPT2, EPT2)], src_v, sem_e)
            cp_d = pltpu.async_copy(
                ei_h.at[pl.ds(Ep + ebase, EPT)], dst_v, sem_e)
            cp_x = pltpu.async_copy(
                x_h.at[pl.ds(0, G)], x0_v.at[pl.ds(0, G)], sem_x)

        zeros = jnp.zeros((_L,), jnp.float32)

        with jax.named_scope("sc_zero"):
            @plsc.parallel_loop(0, Gp // _L, unroll=4)
            def _(i):
                hist_v[pl.ds(i * _L, _L)] = zeros

            cp_s.wait()
            cp_d.wait()

        ones = jnp.ones((_L,), jnp.float32)

        with jax.named_scope("sc_hist"):
            @plsc.parallel_loop(0, EPT // _L, unroll=8)
            def _(i):
                d = dst_v[pl.ds(i * _L, _L)]
                plsc.addupdate_scatter(hist_v, [d], ones)

        with jax.named_scope("sc_hist_comb"):
            pltpu.sync_copy(hist_v, sh_a.at[sid])

            @plsc.parallel_loop(0, Gp // _L, unroll=4)
            def _(i):
                t_v[pl.ds(i * _L, _L)] = zeros
                c_v[pl.ds(i * _L, _L)] = zeros

            plsc.subcore_barrier()
            cps = [pltpu.async_copy(sh_a.at[k, pl.ds(gbase, GPT)],
                                    red_v.at[k], sem_red) for k in range(_NT)]
            for cp in cps:
                cp.wait()

        with jax.named_scope("sc_dinv"):
            def dbody(i, c):
                acc = red_v[0, pl.ds(i * _L, _L)]
                for k in range(1, _NT):
                    acc = acc + red_v[k, pl.ds(i * _L, _L)]
                deg = acc + 1.0
                bits = plsc.bitcast(deg, jnp.int32)
                y = plsc.bitcast(
                    jnp.int32(0x5F3759DF) - lax.shift_right_logical(bits, 1),
                    jnp.float32)
                for _ in range(3):
                    y = y * (1.5 - 0.5 * deg * y * y)
                dinv_v[pl.ds(gbase + i * _L, _L)] = y
                return c
            lax.fori_loop(0, GPT // _L, dbody, 0)

        with jax.named_scope("sc_dinv_comb"):
            pltpu.sync_copy(dinv_v.at[pl.ds(gbase, GPT)],
                            sh_dinv.at[pl.ds(gbase, GPT)])

            @pl.when(cid == 0)
            def _():
                pltpu.sync_copy(dinv_v.at[pl.ds(gbase, GPT)],
                                dinv_h.at[pl.ds(gbase, GPT)])
            plsc.subcore_barrier()
            pltpu.sync_copy(sh_dinv, dinv_v)
            cp_x.wait()

            @pl.when(cid == 1)
            def _():
                pltpu.sync_copy(x0_v.at[pl.ds(gbase, GPT)],
                                x0p_h.at[pl.ds(gbase, GPT)])

        with jax.named_scope("sc_edge"):
            doff = cid * EPT2

            @plsc.parallel_loop(0, EPT2 // _L, unroll=8)
            def _(i):
                s = src_v[pl.ds(i * _L, _L)]
                d = dst_v[pl.ds(doff + i * _L, _L)]
                dv_s = plsc.load_gather(dinv_v, [s])
                dv_d = plsc.load_gather(dinv_v, [d])
                xs = plsc.load_gather(x0_v, [s])
                plsc.addupdate_scatter(t_v, [d], xs * dv_s)
                plsc.addupdate_scatter(c_v, [s], dv_d)

        with jax.named_scope("sc_tc_comb"):
            cpt = pltpu.async_copy(t_v, sh_a.at[sid], sem_e)
            cpc = pltpu.async_copy(c_v, sh_b.at[sid], sem_x)
            cpt.wait()
            cpc.wait()
            plsc.subcore_barrier()
            cps = [pltpu.async_copy(sh_a.at[k, pl.ds(gbase, GPT)],
                                    red_v.at[k], sem_red) for k in range(_NT)]
            cps += [pltpu.async_copy(sh_b.at[k, pl.ds(gbase, GPT)],
                                     red2_v.at[k], sem_out) for k in range(_NT)]
            for cp in cps:
                cp.wait()

        with jax.named_scope("sc_out"):
            def f1body(i, c):
                acc = red_v[0, pl.ds(i * _L, _L)]
                for k in range(1, _NT):
                    acc = acc + red_v[k, pl.ds(i * _L, _L)]
                o1_v[pl.ds(i * _L, _L)] = acc
                return c
            lax.fori_loop(0, GPT // _L, f1body, 0)
            cp1 = pltpu.async_copy(
                o1_v, t2_h.at[pl.ds(cid * Gp + gbase, GPT)], sem_e)

            def f2body(i, c):
                acc = red2_v[0, pl.ds(i * _L, _L)]
                for k in range(1, _NT):
                    acc = acc + red2_v[k, pl.ds(i * _L, _L)]
                o2_v[pl.ds(i * _L, _L)] = acc
                return c
            lax.fori_loop(0, GPT // _L, f2body, 0)
            pltpu.sync_copy(o2_v, c2_h.at[pl.ds(cid * Gp + gbase, GPT)])
            cp1.wait()

    return pl.kernel(
        body,
        out_type=(jax.ShapeDtypeStruct((_NC * Gp,), jnp.float32),
                  jax.ShapeDtypeStruct((_NC * Gp,), jnp.float32),
                  jax.ShapeDtypeStruct((Gp,), jnp.float32),
                  jax.ShapeDtypeStruct((Gp,), jnp.float32)),
        mesh=mesh,
        compiler_params=pltpu.CompilerParams(needs_layout_passes=False),
        scratch_types=[
            pltpu.VMEM((EPT2,), jnp.int32),
            pltpu.VMEM((EPT,), jnp.int32),
            pltpu.VMEM((Gp,), jnp.float32),
            pltpu.VMEM((Gp,), jnp.float32),
            pltpu.VMEM((Gp,), jnp.float32),
            pltpu.VMEM((Gp,), jnp.float32),
            pltpu.VMEM((Gp,), jnp.float32),
            pltpu.VMEM((_NT, GPT), jnp.float32),
            pltpu.VMEM((_NT, GPT), jnp.float32),
            pltpu.VMEM((GPT,), jnp.float32),
            pltpu.VMEM((GPT,), jnp.float32),
            pltpu.SemaphoreType.DMA,
            pltpu.SemaphoreType.DMA,
            pltpu.SemaphoreType.DMA,
            pltpu.SemaphoreType.DMA,
            pltpu.VMEM_SHARED((_NT, Gp), jnp.float32),
            pltpu.VMEM_SHARED((_NT, Gp), jnp.float32),
            pltpu.VMEM_SHARED((Gp,), jnp.float32),
        ],
    )


def _make_rest_kernel(B, G, HID, OUT):
    def body(x_ref, w1_ref, b1_ref, w2_ref, b2_ref, out_ref):
        h = jnp.maximum(w1_ref[...] * x_ref[0] + b1_ref[...], 0.0)
        part = jnp.sum(h, axis=1, keepdims=True)
        out_ref[0] = lax.dot_general(
            part * (1.0 / G), w2_ref[...],
            (((0,), (0,)), ((), ())),
            preferred_element_type=jnp.float32) + b2_ref[...]

    return pl.pallas_call(
        body,
        grid=(B - 1,),
        in_specs=[
            pl.BlockSpec((1, 1, G), lambda i: (i + 1, 0, 0)),
            pl.BlockSpec((HID, 1), lambda i: (0, 0)),
            pl.BlockSpec((HID, 1), lambda i: (0, 0)),
            pl.BlockSpec((HID, OUT), lambda i: (0, 0)),
            pl.BlockSpec((1, OUT), lambda i: (0, 0)),
        ],
        out_specs=pl.BlockSpec((1, 1, OUT), lambda i: (i + 1, 0, 0)),
        out_shape=jax.ShapeDtypeStruct((B, 1, OUT), jnp.float32),
    )


def _make_row0_kernel(G, Gp, HID, OUT, B):
    def body(t2_ref, c2_ref, dinv_ref, x0_ref, w1_ref, b1_ref, w2_ref,
             b2_ref, rest_ref, out_ref):
        del rest_ref
        t = t2_ref[:, :Gp] + t2_ref[:, Gp:]
        c = c2_ref[:, :Gp] + c2_ref[:, Gp:]
        dv = dinv_ref[...]
        s1 = dv * t + dv * dv * x0_ref[...]
        col = lax.broadcasted_iota(jnp.int32, (1, Gp), 1)
        wgt = jnp.where(col < G, dv * c + dv * dv, 0.0)
        h = jnp.maximum(w1_ref[...] * s1 + b1_ref[...], 0.0)
        part = jnp.sum(h * wgt, axis=1, keepdims=True)
        out_ref[0] = lax.dot_general(
            part * (1.0 / G), w2_ref[...],
            (((0,), (0,)), ((), ())),
            preferred_element_type=jnp.float32) + b2_ref[...]

    return pl.pallas_call(
        body,
        grid=(1,),
        in_specs=[
            pl.BlockSpec((1, _NC * Gp), lambda i: (0, 0)),
            pl.BlockSpec((1, _NC * Gp), lambda i: (0, 0)),
            pl.BlockSpec((1, Gp), lambda i: (0, 0)),
            pl.BlockSpec((1, Gp), lambda i: (0, 0)),
            pl.BlockSpec((HID, 1), lambda i: (0, 0)),
            pl.BlockSpec((HID, 1), lambda i: (0, 0)),
            pl.BlockSpec((HID, OUT), lambda i: (0, 0)),
            pl.BlockSpec((1, OUT), lambda i: (0, 0)),
            pl.BlockSpec(memory_space=pl.ANY),
        ],
        out_specs=pl.BlockSpec((1, 1, OUT), lambda i: (0, 0, 0)),
        out_shape=jax.ShapeDtypeStruct((B, 1, OUT), jnp.float32),
        input_output_aliases={8: 0},
    )


def kernel(x, edge_index, W1, b1, W2, b2):
    B, G = x.shape
    E = edge_index.shape[1]
    HID = W1.shape[1]
    OUT = W2.shape[1]

    Gp = -(-G // (_NT * _L)) * (_NT * _L)
    Ep = -(-E // (_NT * _NC * _L)) * (_NT * _NC * _L)

    ei = edge_index
    if Ep != E:
        pad = jnp.full((2, Ep - E), Gp - 1, dtype=edge_index.dtype)
        ei = jnp.concatenate([edge_index, pad], axis=1)

    t2, c2, dinv, x0p = _make_sc_kernel(G, Gp, Ep)(ei.reshape(2 * Ep), x[0])

    w1c = W1.reshape(HID, 1)
    b1c = b1.reshape(HID, 1)
    b2r = b2.reshape(1, OUT)
    out_rest = _make_rest_kernel(B, G, HID, OUT)(
        x.reshape(B, 1, G), w1c, b1c, W2, b2r)
    out = _make_row0_kernel(G, Gp, HID, OUT, B)(
        t2.reshape(1, _NC * Gp), c2.reshape(1, _NC * Gp),
        dinv.reshape(1, Gp), x0p.reshape(1, Gp), w1c, b1c, W2, b2r,
        out_rest)
    return out.reshape(B, OUT)


# speedup vs baseline: 1.0320x; 1.0320x over previous
import jax
import jax.numpy as jnp
from jax import lax
from jax.experimental import pallas as pl
from jax.experimental.pallas import tpu as pltpu
from jax.experimental.pallas import tpu_sc as plsc

_NT = 16
_NC = 2
_L = 16


def _make_sc_kernel(G, Gp, Ep):
    GPT = Gp // _NT
    EPT = Ep // _NT
    EPT2 = EPT // _NC
    mesh = plsc.VectorSubcoreMesh(
        core_axis_name="c", subcore_axis_name="s", num_cores=_NC)

    def body(ei_h, x_h, t2_h, c2_h, dinv_h, x0p_h,
             src_v, dst_v, x0_v, dinv_v, hist_v, t_v, c_v, red_v, red2_v,
             o1_v, o2_v,
             sem_e, sem_x, sem_red, sem_out, sh_a, sh_b, sh_dinv):
        cid = lax.axis_index("c")
        sid = lax.axis_index("s")
        gbase = sid * GPT
        ebase = sid * EPT

        with jax.named_scope("sc_stage"):
            cp_s = pltpu.async_copy(
                ei_h.at[pl.ds(ebase + cid * EPT2, EPT2)], src_v, sem_e)
            cp_d = pltpu.async_copy(
                ei_h.at[pl.ds(Ep + ebase, EPT)], dst_v, sem_e)
            cp_x = pltpu.async_copy(
                x_h.at[pl.ds(0, G)], x0_v.at[pl.ds(0, G)], sem_x)

        zeros = jnp.zeros((_L,), jnp.float32)

        with jax.named_scope("sc_zero"):
            @plsc.parallel_loop(0, Gp // _L, unroll=4)
            def _(i):
                hist_v[pl.ds(i * _L, _L)] = zeros
                t_v[pl.ds(i * _L, _L)] = zeros
                c_v[pl.ds(i * _L, _L)] = zeros

            cp_s.wait()
            cp_d.wait()

        ones = jnp.ones((_L,), jnp.float32)

        with jax.named_scope("sc_hist"):
            @plsc.parallel_loop(0, EPT // _L, unroll=8)
            def _(i):
                d = dst_v[pl.ds(i * _L, _L)]
                plsc.addupdate_scatter(hist_v, [d], ones)

        with jax.named_scope("sc_hist_comb"):
            pltpu.sync_copy(hist_v, sh_a.at[sid])
            plsc.subcore_barrier()
            cps = [pltpu.async_copy(sh_a.at[k, pl.ds(gbase, GPT)],
                                    red_v.at[k], sem_red) for k in range(_NT)]
            for cp in cps:
                cp.wait()

        with jax.named_scope("sc_dinv"):
            def dbody(i, c):
                acc = red_v[0, pl.ds(i * _L, _L)]
                for k in range(1, _NT):
                    acc = acc + red_v[k, pl.ds(i * _L, _L)]
                deg = acc + 1.0
                bits = plsc.bitcast(deg, jnp.int32)
                y = plsc.bitcast(
                    jnp.int32(0x5F3759DF) - lax.shift_right_logical(bits, 1),
                    jnp.float32)
                for _ in range(3):
                    y = y * (1.5 - 0.5 * deg * y * y)
                dinv_v[pl.ds(gbase + i * _L, _L)] = y
                return c
            lax.fori_loop(0, GPT // _L, dbody, 0)

        with jax.named_scope("sc_dinv_comb"):
            pltpu.sync_copy(dinv_v.at[pl.ds(gbase, GPT)],
                            sh_dinv.at[pl.ds(gbase, GPT)])

            @pl.when(cid == 0)
            def _():
                pltpu.sync_copy(dinv_v.at[pl.ds(gbase, GPT)],
                                dinv_h.at[pl.ds(gbase, GPT)])
            plsc.subcore_barrier()
            pltpu.sync_copy(sh_dinv, dinv_v)
            cp_x.wait()

            @pl.when(cid == 1)
            def _():
                pltpu.sync_copy(x0_v.at[pl.ds(gbase, GPT)],
                                x0p_h.at[pl.ds(gbase, GPT)])

        with jax.named_scope("sc_edge"):
            doff = cid * EPT2

            @plsc.parallel_loop(0, EPT2 // _L, unroll=8)
            def _(i):
                s = src_v[pl.ds(i * _L, _L)]
                d = dst_v[pl.ds(doff + i * _L, _L)]
                dv_s = plsc.load_gather(dinv_v, [s])
                dv_d = plsc.load_gather(dinv_v, [d])
                xs = plsc.load_gather(x0_v, [s])
                plsc.addupdate_scatter(t_v, [d], xs * dv_s)
                plsc.addupdate_scatter(c_v, [s], dv_d)

        with jax.named_scope("sc_tc_comb"):
            cpt = pltpu.async_copy(t_v, sh_a.at[sid], sem_e)
            cpc = pltpu.async_copy(c_v, sh_b.at[sid], sem_x)
            cpt.wait()
            cpc.wait()
            plsc.subcore_barrier()
            cps = [pltpu.async_copy(sh_a.at[k, pl.ds(gbase, GPT)],
                                    red_v.at[k], sem_red) for k in range(_NT)]
            cps += [pltpu.async_copy(sh_b.at[k, pl.ds(gbase, GPT)],
                                     red2_v.at[k], sem_out) for k in range(_NT)]
            for cp in cps:
                cp.wait()

        with jax.named_scope("sc_out"):
            def f1body(i, c):
                acc = red_v[0, pl.ds(i * _L, _L)]
                for k in range(1, _NT):
                    acc = acc + red_v[k, pl.ds(i * _L, _L)]
                o1_v[pl.ds(i * _L, _L)] = acc
                return c
            lax.fori_loop(0, GPT // _L, f1body, 0)
            cp1 = pltpu.async_copy(
                o1_v, t2_h.at[pl.ds(cid * Gp + gbase, GPT)], sem_e)

            def f2body(i, c):
                acc = red2_v[0, pl.ds(i * _L, _L)]
                for k in range(1, _NT):
                    acc = acc + red2_v[k, pl.ds(i * _L, _L)]
                o2_v[pl.ds(i * _L, _L)] = acc
                return c
            lax.fori_loop(0, GPT // _L, f2body, 0)
            pltpu.sync_copy(o2_v, c2_h.at[pl.ds(cid * Gp + gbase, GPT)])
            cp1.wait()

    return pl.kernel(
        body,
        out_type=(jax.ShapeDtypeStruct((_NC * Gp,), jnp.float32),
                  jax.ShapeDtypeStruct((_NC * Gp,), jnp.float32),
                  jax.ShapeDtypeStruct((Gp,), jnp.float32),
                  jax.ShapeDtypeStruct((Gp,), jnp.float32)),
        mesh=mesh,
        compiler_params=pltpu.CompilerParams(needs_layout_passes=False),
        scratch_types=[
            pltpu.VMEM((EPT2,), jnp.int32),
            pltpu.VMEM((EPT,), jnp.int32),
            pltpu.VMEM((Gp,), jnp.float32),
            pltpu.VMEM((Gp,), jnp.float32),
            pltpu.VMEM((Gp,), jnp.float32),
            pltpu.VMEM((Gp,), jnp.float32),
            pltpu.VMEM((Gp,), jnp.float32),
            pltpu.VMEM((_NT, GPT), jnp.float32),
            pltpu.VMEM((_NT, GPT), jnp.float32),
            pltpu.VMEM((GPT,), jnp.float32),
            pltpu.VMEM((GPT,), jnp.float32),
            pltpu.SemaphoreType.DMA,
            pltpu.SemaphoreType.DMA,
            pltpu.SemaphoreType.DMA,
            pltpu.SemaphoreType.DMA,
            pltpu.VMEM_SHARED((_NT, Gp), jnp.float32),
            pltpu.VMEM_SHARED((_NT, Gp), jnp.float32),
            pltpu.VMEM_SHARED((Gp,), jnp.float32),
        ],
    )


def _make_rest_kernel(B, G, HID, OUT):
    def body(x_ref, w1_ref, b1_ref, w2_ref, b2_ref, out_ref):
        h = jnp.maximum(w1_ref[...] * x_ref[0] + b1_ref[...], 0.0)
        part = jnp.sum(h, axis=1, keepdims=True)
        out_ref[0] = lax.dot_general(
            part * (1.0 / G), w2_ref[...],
            (((0,), (0,)), ((), ())),
            preferred_element_type=jnp.float32) + b2_ref[...]

    return pl.pallas_call(
        body,
        grid=(B - 1,),
        in_specs=[
            pl.BlockSpec((1, 1, G), lambda i: (i + 1, 0, 0)),
            pl.BlockSpec((HID, 1), lambda i: (0, 0)),
            pl.BlockSpec((HID, 1), lambda i: (0, 0)),
            pl.BlockSpec((HID, OUT), lambda i: (0, 0)),
            pl.BlockSpec((1, OUT), lambda i: (0, 0)),
        ],
        out_specs=pl.BlockSpec((1, 1, OUT), lambda i: (i + 1, 0, 0)),
        out_shape=jax.ShapeDtypeStruct((B, 1, OUT), jnp.float32),
    )


def _make_row0_kernel(G, Gp, HID, OUT, B):
    def body(t2_ref, c2_ref, dinv_ref, x0_ref, w1_ref, b1_ref, w2_ref,
             b2_ref, rest_ref, out_ref):
        del rest_ref
        t = t2_ref[:, :Gp] + t2_ref[:, Gp:]
        c = c2_ref[:, :Gp] + c2_ref[:, Gp:]
        dv = dinv_ref[...]
        s1 = dv * t + dv * dv * x0_ref[...]
        col = lax.broadcasted_iota(jnp.int32, (1, Gp), 1)
        wgt = jnp.where(col < G, dv * c + dv * dv, 0.0)
        h = jnp.maximum(w1_ref[...] * s1 + b1_ref[...], 0.0)
        part = jnp.sum(h * wgt, axis=1, keepdims=True)
        out_ref[0] = lax.dot_general(
            part * (1.0 / G), w2_ref[...],
            (((0,), (0,)), ((), ())),
            preferred_element_type=jnp.float32) + b2_ref[...]

    return pl.pallas_call(
        body,
        grid=(1,),
        in_specs=[
            pl.BlockSpec((1, _NC * Gp), lambda i: (0, 0)),
            pl.BlockSpec((1, _NC * Gp), lambda i: (0, 0)),
            pl.BlockSpec((1, Gp), lambda i: (0, 0)),
            pl.BlockSpec((1, Gp), lambda i: (0, 0)),
            pl.BlockSpec((HID, 1), lambda i: (0, 0)),
            pl.BlockSpec((HID, 1), lambda i: (0, 0)),
            pl.BlockSpec((HID, OUT), lambda i: (0, 0)),
            pl.BlockSpec((1, OUT), lambda i: (0, 0)),
            pl.BlockSpec(memory_space=pl.ANY),
        ],
        out_specs=pl.BlockSpec((1, 1, OUT), lambda i: (0, 0, 0)),
        out_shape=jax.ShapeDtypeStruct((B, 1, OUT), jnp.float32),
        input_output_aliases={8: 0},
    )


def kernel(x, edge_index, W1, b1, W2, b2):
    B, G = x.shape
    E = edge_index.shape[1]
    HID = W1.shape[1]
    OUT = W2.shape[1]

    Gp = -(-G // (_NT * _L)) * (_NT * _L)
    Ep = -(-E // (_NT * _NC * _L)) * (_NT * _NC * _L)

    ei = edge_index
    if Ep != E:
        pad = jnp.full((2, Ep - E), Gp - 1, dtype=edge_index.dtype)
        ei = jnp.concatenate([edge_index, pad], axis=1)

    t2, c2, dinv, x0p = _make_sc_kernel(G, Gp, Ep)(
        ei.reshape(2 * Ep), x.reshape(B * G))

    w1c = W1.reshape(HID, 1)
    b1c = b1.reshape(HID, 1)
    b2r = b2.reshape(1, OUT)
    out_rest = _make_rest_kernel(B, G, HID, OUT)(
        x.reshape(B, 1, G), w1c, b1c, W2, b2r)
    out = _make_row0_kernel(G, Gp, HID, OUT, B)(
        t2.reshape(1, _NC * Gp), c2.reshape(1, _NC * Gp),
        dinv.reshape(1, Gp), x0p.reshape(1, Gp), w1c, b1c, W2, b2r,
        out_rest)
    return out.reshape(B, OUT)
